# baseline (speedup 1.0000x reference)
import functools

import jax
import jax.numpy as jnp
from jax import lax
from jax.experimental import pallas as pl
from jax.experimental.pallas import tpu as pltpu
from jax.experimental.pallas import tpu_sc as plsc

D_MODEL = 64
SCALE = 8.0
CHUNK = 128
LANES = 16
NB = 5
PAIR = 2 * D_MODEL


@functools.partial(jax.jit, static_argnums=(2, 3, 4))
def _run(x_flat, table_pairs, num_cores, num_subcores, b_per_w):
    n_chunks = b_per_w // CHUNK
    n_groups = n_chunks // NB
    nw = num_cores * num_subcores
    B = nw * b_per_w

    mesh = plsc.VectorSubcoreMesh(core_axis_name="c", subcore_axis_name="s")

    @functools.partial(
        pl.kernel,
        mesh=mesh,
        out_type=jax.ShapeDtypeStruct((B // 2, PAIR), jnp.float32),
        scratch_types=[
            pltpu.VMEM((n_chunks, CHUNK), jnp.int32),
            pltpu.VMEM((NB, CHUNK), jnp.int32),
            pltpu.VMEM((NB, CHUNK, PAIR), jnp.float32),
            pltpu.VMEM((NB, CHUNK // 2, PAIR), jnp.float32),
            pltpu.SemaphoreType.DMA((NB,)),
            pltpu.SemaphoreType.DMA((NB,)),
        ],
    )
    def k(x_hbm, table_hbm, out_hbm, idx_v, idx2_v, gbuf, sbuf, gsem, ssem):
        wid = lax.axis_index("s") * num_cores + lax.axis_index("c")
        base2 = wid * (b_per_w // 2)
        pltpu.sync_copy(x_hbm.at[wid], idx_v)

        def g_start(ci, b):
            for j in range(CHUNK // LANES):
                sl = pl.ds(j * LANES, LANES)
                idx2_v[b, sl] = idx_v[ci, sl] >> 1
            pltpu.make_async_copy(
                table_hbm.at[idx2_v.at[b]], gbuf.at[b], gsem.at[b]
            ).start()

        def g_wait(b):
            pltpu.make_async_copy(
                table_hbm.at[idx2_v.at[b]], gbuf.at[b], gsem.at[b]
            ).wait()

        def s_start(ci, b):
            pltpu.make_async_copy(
                sbuf.at[b],
                out_hbm.at[pl.ds(base2 + ci * (CHUNK // 2), CHUNK // 2)],
                ssem.at[b],
            ).start()

        def s_wait(b):
            pltpu.make_async_copy(
                sbuf.at[b], out_hbm.at[pl.ds(base2, CHUNK // 2)], ssem.at[b]
            ).wait()

        def select_scale_chunk(ci, b):
            def sgroup(k, c):
                offv = (idx_v[ci, pl.ds(k * LANES, LANES)] & 1) * D_MODEL
                for l in range(LANES):
                    off = offv[l]
                    for j in range(D_MODEL // LANES):
                        dst = pl.ds((l % 2) * D_MODEL + j * LANES, LANES)
                        src = pl.ds(off + j * LANES, LANES)
                        sbuf[b, k * (LANES // 2) + l // 2, dst] = (
                            gbuf[b, k * LANES + l, src] * SCALE
                        )
                return c

            lax.fori_loop(0, CHUNK // LANES, sgroup, 0)

        for b in range(NB):
            g_start(b, b)

        def group(cg, c):
            for b in range(NB):
                ci = cg * NB + b
                g_wait(b)

                @pl.when(cg > 0)
                def _():
                    s_wait(b)

                select_scale_chunk(ci, b)

                @pl.when(cg < n_groups - 1)
                def _():
                    g_start(ci + NB, b)

                s_start(ci, b)
            return c

        lax.fori_loop(0, n_groups, group, 0)

        for b in range(NB):
            s_wait(b)

    return k(x_flat.reshape(nw, n_chunks, CHUNK), table_pairs)


def kernel(x, table):
    B0, B1 = x.shape
    B = B0 * B1
    info = plsc.get_sparse_core_info()
    nw = info.num_cores * info.num_subcores
    b_per_w = B // nw
    table_pairs = table.reshape(table.shape[0] // 2, PAIR)
    out = _run(
        x.reshape(B), table_pairs, info.num_cores, info.num_subcores, b_per_w
    )
    return out.reshape(B0, B1, D_MODEL)


# speedup vs baseline: 1.5476x; 1.5476x over previous
import functools

import jax
import jax.numpy as jnp
from jax import lax
from jax.experimental import pallas as pl
from jax.experimental.pallas import tpu as pltpu
from jax.experimental.pallas import tpu_sc as plsc

D_MODEL = 64
SCALE = 8.0
CHUNK = 128
LANES = 16
NB = 5
PAIR = 2 * D_MODEL


@functools.partial(jax.jit, static_argnums=(2, 3, 4))
def _run(x_flat, table, num_cores, num_subcores, b_per_w):
    n_chunks = b_per_w // CHUNK
    n_groups = n_chunks // NB
    nw = num_cores * num_subcores
    B = nw * b_per_w
    hc = CHUNK // 2

    mesh = plsc.VectorSubcoreMesh(core_axis_name="c", subcore_axis_name="s")
    n_chunks_pad = ((n_chunks + 7) // 8) * 8

    @functools.partial(
        pl.kernel,
        mesh=mesh,
        out_type=jax.ShapeDtypeStruct((B // 2, PAIR), jnp.float32),
        scratch_types=[
            pltpu.VMEM((n_chunks_pad, CHUNK), jnp.int32),
            pltpu.VMEM((NB, hc, PAIR), jnp.float32),
            pltpu.VMEM((NB, hc, PAIR), jnp.float32),
            pltpu.SemaphoreType.DMA((NB,)),
            pltpu.SemaphoreType.DMA((NB,)),
        ],
    )
    def k(x_hbm, table_hbm, out_hbm, idx_v, gbuf, sbuf, gsem, ssem):
        wid = lax.axis_index("s") * num_cores + lax.axis_index("c")
        base2 = wid * (b_per_w // 2)
        pltpu.sync_copy(x_hbm.at[pl.ds(wid * n_chunks_pad, n_chunks_pad)], idx_v)

        def g_start(ci, b):
            def ggroup(k_, c):
                rv = idx_v[ci, pl.ds(k_ * LANES, LANES)]
                for l in range(LANES):
                    slot = k_ * (LANES // 2) + l // 2
                    half = pl.ds((l % 2) * D_MODEL, D_MODEL)
                    pltpu.make_async_copy(
                        table_hbm.at[rv[l]],
                        gbuf.at[b, slot, half],
                        gsem.at[b],
                    ).start()
                return c

            lax.fori_loop(0, CHUNK // LANES, ggroup, 0)

        def g_wait(b):
            pltpu.make_async_copy(
                out_hbm.at[pl.ds(0, hc)], gbuf.at[b], gsem.at[b]
            ).wait()

        def s_start(ci, b):
            pltpu.make_async_copy(
                sbuf.at[b],
                out_hbm.at[pl.ds(base2 + ci * hc, hc)],
                ssem.at[b],
            ).start()

        def s_wait(b):
            pltpu.make_async_copy(
                sbuf.at[b], out_hbm.at[pl.ds(base2, hc)], ssem.at[b]
            ).wait()

        def scale_chunk(b):
            def srow(r, c):
                for j in range(PAIR // LANES):
                    sl = pl.ds(j * LANES, LANES)
                    sbuf[b, r, sl] = gbuf[b, r, sl] * SCALE
                return c

            lax.fori_loop(0, hc, srow, 0, unroll=2)

        for b in range(NB):
            g_start(b, b)

        def group(cg, c):
            for b in range(NB):
                ci = cg * NB + b
                g_wait(b)

                @pl.when(cg > 0)
                def _():
                    s_wait(b)

                scale_chunk(b)

                @pl.when(cg < n_groups - 1)
                def _():
                    g_start(ci + NB, b)

                s_start(ci, b)
            return c

        lax.fori_loop(0, n_groups, group, 0)

        for b in range(NB):
            s_wait(b)

    xr = x_flat.reshape(nw, n_chunks, CHUNK)
    xp = jnp.pad(xr, ((0, 0), (0, n_chunks_pad - n_chunks), (0, 0)))
    return k(xp.reshape(nw * n_chunks_pad, CHUNK), table)


def kernel(x, table):
    B0, B1 = x.shape
    B = B0 * B1
    info = plsc.get_sparse_core_info()
    nw = info.num_cores * info.num_subcores
    b_per_w = B // nw
    out = _run(x.reshape(B), table, info.num_cores, info.num_subcores, b_per_w)
    return out.reshape(B0, B1, D_MODEL)
